# baseline (speedup 1.0000x reference)
import functools

import jax
import jax.numpy as jnp
from jax import lax
from jax.experimental import pallas as pl
from jax.experimental.pallas import tpu as pltpu
from jax.experimental.pallas import tpu_sc as plsc

_NC = 2
_NS = 16
_NW = _NC * _NS
_L = 16


@functools.partial(jax.jit, static_argnames=("nb", "hist", "dim"))
def _gather_rows(idx_flat, table_flat, nb, hist, dim):
    vocab_words = table_flat.shape[0]
    n_dt = dim // 8
    n_hc = (hist + 3) // 4
    mesh = plsc.VectorSubcoreMesh(core_axis_name="c", subcore_axis_name="s")

    @functools.partial(
        pl.kernel,
        mesh=mesh,
        compiler_params=pltpu.CompilerParams(needs_layout_passes=False),
        out_type=jax.ShapeDtypeStruct((hist, dim, nb), jnp.float32),
        scratch_types=[
            pltpu.VMEM((vocab_words,), jnp.float32),
            pltpu.VMEM((nb,), jnp.int32),
            pltpu.VMEM((nb,), jnp.int32),
            pltpu.VMEM((8, nb), jnp.float32),
            pltpu.VMEM((8, nb), jnp.float32),
            pltpu.SemaphoreType.DMA,
            pltpu.SemaphoreType.DMA,
            pltpu.SemaphoreType.DMA,
            pltpu.SemaphoreType.DMA,
            pltpu.SemaphoreType.DMA,
        ],
    )
    def k(idx_hbm, table_hbm, out_hbm, tbl_v, idx0, idx1, st0, st1,
          tsem, isem0, isem1, osem0, osem1):
        idxb = (idx0, idx1)
        stb = (st0, st1)
        isems = (isem0, isem1)
        osems = (osem0, osem1)
        wid = lax.axis_index("s") * _NC + lax.axis_index("c")
        dt = wid % n_dt
        hc = wid // n_dt
        d0 = pl.multiple_of(dt * 8, 8)

        ct = pltpu.async_copy(table_hbm, tbl_v, tsem)

        def h_of(j):
            return jnp.minimum(hc + 4 * j, hist - 1)

        def fire_idx(j):
            r = j % 2
            return pltpu.async_copy(
                idx_hbm.at[pl.ds(h_of(j) * nb, nb)], idxb[r], isems[r])

        ic = {0: fire_idx(0), 1: fire_idx(1)}
        ct.wait()

        o = [None, None]
        for j in range(n_hc):
            r = j % 2
            ic[j].wait()
            if o[r] is not None:
                o[r].wait()
            iv = idxb[r]
            st = stb[r]

            @plsc.parallel_loop(0, nb // _L, 1, unroll=4)
            def grp(g, iv=iv, st=st):
                bidx = iv[pl.ds(g * _L, _L)]
                base = bidx * dim + d0
                for dsub in range(8):
                    st[dsub, pl.ds(g * _L, _L)] = plsc.load_gather(
                        tbl_v, [base + dsub])
            if j + 2 < n_hc:
                ic[j + 2] = fire_idx(j + 2)
            o[r] = pltpu.async_copy(
                st, out_hbm.at[h_of(j), pl.ds(d0, 8)], osems[r])
        for r in range(2):
            if o[r] is not None:
                o[r].wait()

    return k(idx_flat, table_flat)


def kernel(tensor, weight):
    nb, hist = tensor.shape
    vocab, dim = weight.shape
    idx_flat = tensor.astype(jnp.int32).T.reshape(-1)
    table_flat = weight.reshape(-1)
    out = _gather_rows(idx_flat, table_flat, nb=nb, hist=hist, dim=dim)
    return out.transpose(2, 0, 1)


# speedup vs baseline: 2.0376x; 2.0376x over previous
import functools

import jax
import jax.numpy as jnp
from jax import lax
from jax.experimental import pallas as pl
from jax.experimental.pallas import tpu as pltpu
from jax.experimental.pallas import tpu_sc as plsc

_NC = 2
_NS = 16
_NW = _NC * _NS
_L = 16


@functools.partial(jax.jit, static_argnames=("nb", "hist", "dim"))
def _gather_rows(idx_flat, table_flat, nb, hist, dim):
    vocab_words = table_flat.shape[0]
    n_dt = dim // 8
    n_hc = (hist + 3) // 4
    mesh = plsc.VectorSubcoreMesh(core_axis_name="c", subcore_axis_name="s")

    @functools.partial(
        pl.kernel,
        mesh=mesh,
        compiler_params=pltpu.CompilerParams(needs_layout_passes=False),
        out_type=jax.ShapeDtypeStruct((hist, dim, nb), jnp.float32),
        scratch_types=[
            pltpu.VMEM((vocab_words,), jnp.float32),
            pltpu.VMEM((nb,), jnp.int32),
            pltpu.VMEM((nb,), jnp.int32),
            pltpu.VMEM((8, nb), jnp.float32),
            pltpu.VMEM((8, nb), jnp.float32),
            pltpu.SemaphoreType.DMA,
            pltpu.SemaphoreType.DMA,
            pltpu.SemaphoreType.DMA,
            pltpu.SemaphoreType.DMA,
            pltpu.SemaphoreType.DMA,
        ],
    )
    def k(idx_hbm, table_hbm, out_hbm, tbl_v, idx0, idx1, st0, st1,
          tsem, isem0, isem1, osem0, osem1):
        idxb = (idx0, idx1)
        stb = (st0, st1)
        isems = (isem0, isem1)
        osems = (osem0, osem1)
        wid = lax.axis_index("s") * _NC + lax.axis_index("c")
        dt = wid % n_dt
        hc = wid // n_dt
        d0 = pl.multiple_of(dt * 8, 8)
        vocab = vocab_words // dim

        ct = pltpu.async_copy(table_hbm, tbl_v, tsem)

        def h_of(j):
            return jnp.minimum(hc + 4 * j, hist - 1)

        def fire_idx(j):
            r = j % 2
            return pltpu.async_copy(
                idx_hbm.at[pl.ds(h_of(j) * nb, nb)], idxb[r], isems[r])

        ic = {0: fire_idx(0), 1: fire_idx(1)}
        ct.wait()

        o = [None, None]
        for j in range(n_hc):
            r = j % 2
            ic[j].wait()
            if o[r] is not None:
                o[r].wait()
            iv = idxb[r]
            st = stb[r]

            @plsc.parallel_loop(0, nb // _L, 1, unroll=2)
            def grp(g, iv=iv, st=st):
                bidx = iv[pl.ds(g * _L, _L)]
                for dsub in range(8):
                    st[dsub, pl.ds(g * _L, _L)] = plsc.load_gather(
                        tbl_v, [bidx + (d0 + dsub) * vocab])
            if j + 2 < n_hc:
                ic[j + 2] = fire_idx(j + 2)
            o[r] = pltpu.async_copy(
                st, out_hbm.at[h_of(j), pl.ds(d0, 8)], osems[r])
        for r in range(2):
            if o[r] is not None:
                o[r].wait()

    return k(idx_flat, table_flat)


def kernel(tensor, weight):
    nb, hist = tensor.shape
    vocab, dim = weight.shape
    idx_flat = tensor.astype(jnp.int32).T.reshape(-1)
    table_flat = weight.T.reshape(-1)
    out = _gather_rows(idx_flat, table_flat, nb=nb, hist=hist, dim=dim)
    return out.transpose(2, 0, 1)


# speedup vs baseline: 2.1538x; 1.0571x over previous
import functools

import jax
import jax.numpy as jnp
from jax import lax
from jax.experimental import pallas as pl
from jax.experimental.pallas import tpu as pltpu
from jax.experimental.pallas import tpu_sc as plsc

_NC = 2
_NS = 16
_NW = _NC * _NS
_L = 16


@functools.partial(jax.jit, static_argnames=("nb", "hist", "dim"))
def _gather_rows(idx_flat, table_t, nb, hist, dim):
    vocab = table_t.shape[1]
    n_dt = dim // 8
    n_hc = -(-hist // 4)
    n_ch = 2 * (-(-n_hc // 2))
    mesh = plsc.VectorSubcoreMesh(core_axis_name="c", subcore_axis_name="s")

    @functools.partial(
        pl.kernel,
        mesh=mesh,
        compiler_params=pltpu.CompilerParams(needs_layout_passes=False),
        out_type=jax.ShapeDtypeStruct((hist, dim, nb), jnp.float32),
        scratch_types=[
            pltpu.VMEM((8, vocab), jnp.float32),
            pltpu.VMEM((nb,), jnp.int32),
            pltpu.VMEM((nb,), jnp.int32),
            pltpu.VMEM((8, nb), jnp.float32),
            pltpu.VMEM((8, nb), jnp.float32),
            pltpu.SemaphoreType.DMA,
            pltpu.SemaphoreType.DMA,
            pltpu.SemaphoreType.DMA,
            pltpu.SemaphoreType.DMA,
            pltpu.SemaphoreType.DMA,
        ],
    )
    def k(idx_hbm, table_hbm, out_hbm, tbl_v, idx0, idx1, st0, st1,
          tsem, isem0, isem1, osem0, osem1):
        idxb = (idx0, idx1)
        stb = (st0, st1)
        isems = (isem0, isem1)
        osems = (osem0, osem1)
        wid = lax.axis_index("s") * _NC + lax.axis_index("c")
        dt = wid % n_dt
        hc = wid // n_dt
        d0 = pl.multiple_of(dt * 8, 8)

        ct = pltpu.async_copy(table_hbm.at[pl.ds(d0, 8)], tbl_v, tsem)

        def h_of(j):
            return jnp.minimum(hc + 4 * j, hist - 1)

        def fire_idx(j, r):
            return pltpu.async_copy(
                idx_hbm.at[pl.ds(h_of(j) * nb, nb)], idxb[r], isems[r])

        def wait_idx(j, r):
            pltpu.make_async_copy(
                idx_hbm.at[pl.ds(h_of(j) * nb, nb)], idxb[r], isems[r]).wait()

        def fire_out(j, r):
            return pltpu.async_copy(
                stb[r], out_hbm.at[h_of(j), pl.ds(d0, 8)], osems[r])

        def wait_out(j, r):
            pltpu.make_async_copy(
                stb[r], out_hbm.at[h_of(j), pl.ds(d0, 8)], osems[r]).wait()

        def compute(j, r):
            iv = idxb[r]
            st = stb[r]
            rows = [jnp.full((_L,), s, jnp.int32) for s in range(8)]

            @plsc.parallel_loop(0, nb // _L, 1, unroll=2)
            def grp(g):
                bidx = iv[pl.ds(g * _L, _L)]
                for dsub in range(8):
                    st[dsub, pl.ds(g * _L, _L)] = plsc.load_gather(
                        tbl_v, [rows[dsub], bidx])

        for r in (0, 1):
            fire_idx(r, r)
        fire_idx(2, 0)
        fire_idx(3, 1)
        ct.wait()
        for r in (0, 1):
            wait_idx(r, r)
            compute(r, r)
            fire_out(r, r)

        def body(jj, _):
            j0 = 2 * jj
            for r in (0, 1):
                j = j0 + r
                wait_idx(j, r)
                wait_out(j - 2, r)
                compute(j, r)
                fire_idx(j + 2, r)
                fire_out(j, r)
            return 0

        lax.fori_loop(1, n_ch // 2, body, 0)
        for r in (0, 1):
            wait_idx(n_ch + r, r)
            wait_out(n_ch - 2 + r, r)

    return k(idx_flat, table_t)


def kernel(tensor, weight):
    nb, hist = tensor.shape
    dim = weight.shape[1]
    idx_flat = tensor.astype(jnp.int32).T.reshape(-1)
    table_t = weight.T
    out = _gather_rows(idx_flat, table_t, nb=nb, hist=hist, dim=dim)
    return out.transpose(2, 0, 1)
